# baseline (speedup 1.0000x reference)
import functools

import jax
import jax.numpy as jnp
from jax import lax
from jax.experimental import pallas as pl
from jax.experimental.pallas import tpu as pltpu
from jax.experimental.pallas import tpu_sc as plsc

N = 10000
NPAD = 10240
F = 128
H = 128
C = 64
E = 320000
NC = 2
NS = 16
NW = NC * NS
B = 128
CH = E // B
GB = 8
NGRP = CH // GB
GPW = NGRP // NW
XG = NGRP - GPW * NW
TAIL0 = NGRP * GB
TW = CH - TAIL0
ZR = NPAD // NS


def _sc_mesh():
    return plsc.VectorSubcoreMesh(core_axis_name="c", subcore_axis_name="s")


@functools.partial(
    pl.kernel,
    out_type=jax.ShapeDtypeStruct((NC, NPAD, 128), jnp.float32),
    mesh=_sc_mesh(),
    scratch_types=[
        pltpu.VMEM((GB, B), jnp.int32),
        pltpu.VMEM((GB, B), jnp.int32),
        pltpu.VMEM((B,), jnp.int32),
        pltpu.VMEM((B,), jnp.int32),
        pltpu.VMEM((B, 128), jnp.float32),
        pltpu.VMEM((B, 128), jnp.float32),
        pltpu.VMEM_SHARED((NPAD, 128), jnp.float32),
        pltpu.SemaphoreType.DMA,
        pltpu.SemaphoreType.DMA,
        pltpu.SemaphoreType.DMA,
        pltpu.SemaphoreType.DMA,
    ],
)
def _deg_kernel(src_hbm, dst_hbm, st_hbm, dt_hbm, out_hbm, sgrp, dgrp,
                stail, dtail, onesa, onesb, acc, sem0, sem1, sem2, sem3):
    c = lax.axis_index("c")
    s = lax.axis_index("s")
    wid = c * NS + s
    gbase = wid * GPW + jnp.minimum(wid, XG)
    lane = lax.iota(jnp.int32, 16)
    va = jnp.where(lane == 0, 1.0, 0.0).astype(jnp.float32)
    vb = jnp.where(lane == 1, 1.0, 0.0).astype(jnp.float32)
    zv = jnp.zeros((16,), jnp.float32)

    @pl.loop(0, B)
    def _zfill(i):
        for j in range(8):
            onesa[i, pl.ds(j * 16, 16)] = zv
            onesb[i, pl.ds(j * 16, 16)] = zv

    for k in range(ZR // B):
        pltpu.sync_copy(onesa, acc.at[pl.ds(s * ZR + k * B, B)])

    @pl.loop(0, B)
    def _fill(i):
        onesa[i, pl.ds(0, 16)] = va
        onesb[i, pl.ds(0, 16)] = vb

    plsc.subcore_barrier()

    def _do_group(row0):
        pltpu.sync_copy(src_hbm.at[pl.ds(row0, GB)], sgrp)
        pltpu.sync_copy(dst_hbm.at[pl.ds(row0, GB)], dgrp)

        @pl.loop(0, GB // 2)
        def _pair(q):
            h0 = pltpu.async_copy(onesa, acc.at[sgrp.at[2 * q]], sem0,
                                  add=True)
            h1 = pltpu.async_copy(onesb, acc.at[dgrp.at[2 * q]], sem1,
                                  add=True)
            h2 = pltpu.async_copy(onesa, acc.at[sgrp.at[2 * q + 1]], sem2,
                                  add=True)
            h3 = pltpu.async_copy(onesb, acc.at[dgrp.at[2 * q + 1]], sem3,
                                  add=True)
            h0.wait()
            h1.wait()
            h2.wait()
            h3.wait()

    @pl.loop(0, GPW)
    def _group(g):
        _do_group((gbase + g) * GB)

    @pl.when(wid < XG)
    def _extra():
        _do_group((gbase + GPW) * GB)

    @pl.when(wid >= NW - TW)
    def _tail():
        t = (wid - (NW - TW)) * B
        pltpu.sync_copy(st_hbm.at[pl.ds(t, B)], stail)
        pltpu.sync_copy(dt_hbm.at[pl.ds(t, B)], dtail)
        pltpu.sync_copy(onesa, acc.at[stail], add=True)
        pltpu.sync_copy(onesb, acc.at[dtail], add=True)

    plsc.subcore_barrier()

    @pl.when(s == 0)
    def _():
        pltpu.sync_copy(acc, out_hbm.at[c])


def _make_agg(D):
    @functools.partial(
        pl.kernel,
        out_type=jax.ShapeDtypeStruct((NC, NPAD, D), jnp.float32),
        mesh=_sc_mesh(),
        scratch_types=[
            pltpu.VMEM((GB * B,), jnp.int32),
            pltpu.VMEM((GB, B), jnp.int32),
            pltpu.VMEM((B,), jnp.int32),
            pltpu.VMEM((B, D), jnp.float32),
            pltpu.VMEM((B, D), jnp.float32),
            pltpu.VMEM_SHARED((NPAD, D), jnp.float32),
            pltpu.SemaphoreType.DMA,
            pltpu.SemaphoreType.DMA,
            pltpu.SemaphoreType.DMA,
            pltpu.SemaphoreType.DMA,
        ],
    )
    def agg(tab_hbm, src_hbm, dst_hbm, dt_hbm, out_hbm, sgrp, dgrp, dtail,
            rowsa, rowsb, acc, gsa, gsb, ssa, ssb):
        c = lax.axis_index("c")
        s = lax.axis_index("s")
        wid = c * NS + s
        gbase = wid * GPW + jnp.minimum(wid, XG)
        zv = jnp.zeros((16,), jnp.float32)

        @pl.loop(0, B)
        def _zero(i):
            for j in range(D // 16):
                rowsa[i, pl.ds(j * 16, 16)] = zv

        for k in range(ZR // B):
            pltpu.sync_copy(rowsa, acc.at[pl.ds(s * ZR + k * B, B)])
        plsc.subcore_barrier()

        def _do_group(row0):
            pltpu.sync_copy(src_hbm.at[pl.ds(row0 * B, GB * B)], sgrp)
            pltpu.sync_copy(dst_hbm.at[pl.ds(row0, GB)], dgrp)

            @pl.loop(0, GB // 2)
            def _pair(q):
                off = 2 * q * B
                ga = pltpu.async_copy(tab_hbm.at[sgrp.at[pl.ds(off, B)]],
                                      rowsa, gsa)
                gb = pltpu.async_copy(tab_hbm.at[sgrp.at[pl.ds(off + B, B)]],
                                      rowsb, gsb)
                ga.wait()
                ha = pltpu.async_copy(rowsa, acc.at[dgrp.at[2 * q]], ssa,
                                      add=True)
                gb.wait()
                hb = pltpu.async_copy(rowsb, acc.at[dgrp.at[2 * q + 1]], ssb,
                                      add=True)
                ha.wait()
                hb.wait()

        @pl.loop(0, GPW)
        def _group(g):
            _do_group((gbase + g) * GB)

        @pl.when(wid < XG)
        def _extra():
            _do_group((gbase + GPW) * GB)

        @pl.when(wid >= NW - TW)
        def _tail():
            row = TAIL0 + wid - (NW - TW)
            t = (wid - (NW - TW)) * B
            pltpu.sync_copy(src_hbm.at[pl.ds(row * B, B)],
                            sgrp.at[pl.ds(0, B)])
            pltpu.sync_copy(dt_hbm.at[pl.ds(t, B)], dtail)
            pltpu.async_copy(tab_hbm.at[sgrp.at[pl.ds(0, B)]], rowsa,
                             gsa).wait()
            pltpu.sync_copy(rowsa, acc.at[dtail], add=True)

        plsc.subcore_barrier()

        @pl.when(s == 0)
        def _():
            pltpu.sync_copy(acc, out_hbm.at[c])

    return agg


_agg_h = _make_agg(H)


def _inv(col):
    return jnp.where(col > 0, lax.rsqrt(jnp.maximum(col, 1.0)), 0.0)


def _mm1_body(x_ref, w_ref, o_ref):
    o_ref[...] = jnp.dot(
        x_ref[...], w_ref[...], preferred_element_type=jnp.float32
    )


def _scale_body(h_ref, deg_ref, o_ref):
    d = deg_ref[0] + deg_ref[1]
    o_ref[...] = h_ref[...] * _inv(d[:, 0:1])


def _mm2_body(a_ref, deg_ref, w_ref, o_ref):
    d = deg_ref[0] + deg_ref[1]
    inv_out = _inv(d[:, 0:1])
    inv_in = _inv(d[:, 1:2])
    h = jnp.maximum((a_ref[0] + a_ref[1]) * inv_in, 0.0)
    o_ref[...] = (
        jnp.dot(h, w_ref[...], preferred_element_type=jnp.float32) * inv_out
    )


def _fin_body(a_ref, deg_ref, o_ref):
    d = deg_ref[0] + deg_ref[1]
    inv_in = _inv(d[:, 1:2])
    o_ref[...] = (a_ref[0, :, :C] + a_ref[1, :, :C]) * inv_in


R1 = 1000
R2 = 1024


def kernel(x, edge_index, W1, W2):
    src = edge_index[0]
    dst = edge_index[1]
    src2d = src.reshape(CH, B)
    dst2d = dst.reshape(CH, B)
    stl = lax.slice(src, (TAIL0 * B,), (E,))
    dtl = lax.slice(dst, (TAIL0 * B,), (E,))

    degp = _deg_kernel(src2d, dst2d, stl, dtl)
    xw1 = pl.pallas_call(
        _mm1_body,
        grid=(N // R1,),
        in_specs=[
            pl.BlockSpec((R1, F), lambda i: (i, 0)),
            pl.BlockSpec((F, H), lambda i: (0, 0)),
        ],
        out_specs=pl.BlockSpec((R1, H), lambda i: (i, 0)),
        out_shape=jax.ShapeDtypeStruct((N, H), jnp.float32),
    )(x, W1)

    h1s = pl.pallas_call(
        _scale_body,
        grid=(N // R1,),
        in_specs=[
            pl.BlockSpec((R1, H), lambda i: (i, 0)),
            pl.BlockSpec((NC, R1, 128), lambda i: (0, i, 0)),
        ],
        out_specs=pl.BlockSpec((R1, H), lambda i: (i, 0)),
        out_shape=jax.ShapeDtypeStruct((N, H), jnp.float32),
    )(xw1, degp)

    agg1 = _agg_h(h1s, src, dst2d, dtl)

    W2p = jnp.concatenate([W2, jnp.zeros((H, H - C), jnp.float32)], axis=1)
    h2s = pl.pallas_call(
        _mm2_body,
        grid=(NPAD // R2,),
        in_specs=[
            pl.BlockSpec((NC, R2, H), lambda i: (0, i, 0)),
            pl.BlockSpec((NC, R2, 128), lambda i: (0, i, 0)),
            pl.BlockSpec((H, H), lambda i: (0, 0)),
        ],
        out_specs=pl.BlockSpec((R2, H), lambda i: (i, 0)),
        out_shape=jax.ShapeDtypeStruct((NPAD, H), jnp.float32),
    )(agg1, degp, W2p)

    agg2 = _agg_h(h2s, src, dst2d, dtl)

    outp = pl.pallas_call(
        _fin_body,
        grid=(NPAD // R2,),
        in_specs=[
            pl.BlockSpec((NC, R2, H), lambda i: (0, i, 0)),
            pl.BlockSpec((NC, R2, 128), lambda i: (0, i, 0)),
        ],
        out_specs=pl.BlockSpec((R2, C), lambda i: (i, 0)),
        out_shape=jax.ShapeDtypeStruct((NPAD, C), jnp.float32),
    )(agg2, degp)

    return outp[:N]


# speedup vs baseline: 1.0025x; 1.0025x over previous
import functools

import jax
import jax.numpy as jnp
from jax import lax
from jax.experimental import pallas as pl
from jax.experimental.pallas import tpu as pltpu
from jax.experimental.pallas import tpu_sc as plsc

N = 10000
NPAD = 10240
F = 128
H = 128
C = 64
E = 320000
NC = 2
NS = 16
NW = NC * NS
B = 128
CH = E // B
GB = 8
NGRP = CH // GB
GPW = NGRP // NW
XG = NGRP - GPW * NW
TAIL0 = NGRP * GB
TW = CH - TAIL0
ZR = NPAD // NS


def _sc_mesh():
    return plsc.VectorSubcoreMesh(core_axis_name="c", subcore_axis_name="s")


@functools.partial(
    pl.kernel,
    out_type=jax.ShapeDtypeStruct((NC, NPAD, 128), jnp.float32),
    mesh=_sc_mesh(),
    scratch_types=[
        pltpu.VMEM((GB, B), jnp.int32),
        pltpu.VMEM((GB, B), jnp.int32),
        pltpu.VMEM((B,), jnp.int32),
        pltpu.VMEM((B,), jnp.int32),
        pltpu.VMEM((B, 128), jnp.float32),
        pltpu.VMEM((B, 128), jnp.float32),
        pltpu.VMEM_SHARED((NPAD, 128), jnp.float32),
        pltpu.SemaphoreType.DMA,
        pltpu.SemaphoreType.DMA,
        pltpu.SemaphoreType.DMA,
        pltpu.SemaphoreType.DMA,
    ],
)
def _deg_kernel(src_hbm, dst_hbm, st_hbm, dt_hbm, out_hbm, sgrp, dgrp,
                stail, dtail, onesa, onesb, acc, sem0, sem1, sem2, sem3):
    c = lax.axis_index("c")
    s = lax.axis_index("s")
    wid = c * NS + s
    gbase = wid * GPW + jnp.minimum(wid, XG)
    lane = lax.iota(jnp.int32, 16)
    va = jnp.where(lane == 0, 1.0, 0.0).astype(jnp.float32)
    vb = jnp.where(lane == 1, 1.0, 0.0).astype(jnp.float32)
    zv = jnp.zeros((16,), jnp.float32)

    @pl.loop(0, B)
    def _zfill(i):
        for j in range(8):
            onesa[i, pl.ds(j * 16, 16)] = zv
            onesb[i, pl.ds(j * 16, 16)] = zv

    for k in range(ZR // B):
        pltpu.sync_copy(onesa, acc.at[pl.ds(s * ZR + k * B, B)])

    @pl.loop(0, B)
    def _fill(i):
        onesa[i, pl.ds(0, 16)] = va
        onesb[i, pl.ds(0, 16)] = vb

    plsc.subcore_barrier()

    def _do_group(row0):
        pltpu.sync_copy(src_hbm.at[pl.ds(row0, GB)], sgrp)
        pltpu.sync_copy(dst_hbm.at[pl.ds(row0, GB)], dgrp)

        @pl.loop(0, GB // 2)
        def _pair(q):
            h0 = pltpu.async_copy(onesa, acc.at[sgrp.at[2 * q]], sem0,
                                  add=True)
            h1 = pltpu.async_copy(onesb, acc.at[dgrp.at[2 * q]], sem1,
                                  add=True)
            h2 = pltpu.async_copy(onesa, acc.at[sgrp.at[2 * q + 1]], sem2,
                                  add=True)
            h3 = pltpu.async_copy(onesb, acc.at[dgrp.at[2 * q + 1]], sem3,
                                  add=True)
            h0.wait()
            h1.wait()
            h2.wait()
            h3.wait()

    @pl.loop(0, GPW)
    def _group(g):
        _do_group((gbase + g) * GB)

    @pl.when(wid < XG)
    def _extra():
        _do_group((gbase + GPW) * GB)

    @pl.when(wid >= NW - TW)
    def _tail():
        t = (wid - (NW - TW)) * B
        pltpu.sync_copy(st_hbm.at[pl.ds(t, B)], stail)
        pltpu.sync_copy(dt_hbm.at[pl.ds(t, B)], dtail)
        pltpu.sync_copy(onesa, acc.at[stail], add=True)
        pltpu.sync_copy(onesb, acc.at[dtail], add=True)

    plsc.subcore_barrier()

    @pl.when(s == 0)
    def _():
        pltpu.sync_copy(acc, out_hbm.at[c])


def _make_agg(D):
    @functools.partial(
        pl.kernel,
        out_type=jax.ShapeDtypeStruct((NC, NPAD, D), jnp.float32),
        mesh=_sc_mesh(),
        scratch_types=[
            pltpu.VMEM((GB * B,), jnp.int32),
            pltpu.VMEM((GB, B), jnp.int32),
            pltpu.VMEM((B,), jnp.int32),
            pltpu.VMEM((B, D), jnp.float32),
            pltpu.VMEM((B, D), jnp.float32),
            pltpu.VMEM_SHARED((NPAD, D), jnp.float32),
            pltpu.SemaphoreType.DMA,
            pltpu.SemaphoreType.DMA,
            pltpu.SemaphoreType.DMA,
            pltpu.SemaphoreType.DMA,
        ],
    )
    def agg(tab_hbm, src_hbm, dst_hbm, dt_hbm, out_hbm, sgrp, dgrp, dtail,
            rowsa, rowsb, acc, gsa, gsb, ssa, ssb):
        c = lax.axis_index("c")
        s = lax.axis_index("s")
        wid = c * NS + s
        gbase = wid * GPW + jnp.minimum(wid, XG)
        zv = jnp.zeros((16,), jnp.float32)

        @pl.loop(0, B)
        def _zero(i):
            for j in range(D // 16):
                rowsa[i, pl.ds(j * 16, 16)] = zv

        for k in range(ZR // B):
            pltpu.sync_copy(rowsa, acc.at[pl.ds(s * ZR + k * B, B)])
        plsc.subcore_barrier()

        def _do_group(row0):
            pltpu.sync_copy(src_hbm.at[pl.ds(row0 * B, GB * B)], sgrp)
            pltpu.sync_copy(dst_hbm.at[pl.ds(row0, GB)], dgrp)

            @pl.loop(0, GB // 2)
            def _pair(q):
                off = 2 * q * B
                ga = pltpu.async_copy(tab_hbm.at[sgrp.at[pl.ds(off, B)]],
                                      rowsa, gsa)
                gb = pltpu.async_copy(tab_hbm.at[sgrp.at[pl.ds(off + B, B)]],
                                      rowsb, gsb)
                ga.wait()
                ha = pltpu.async_copy(rowsa, acc.at[dgrp.at[2 * q]], ssa,
                                      add=True)
                gb.wait()
                hb = pltpu.async_copy(rowsb, acc.at[dgrp.at[2 * q + 1]], ssb,
                                      add=True)
                ha.wait()
                hb.wait()

        @pl.loop(0, GPW)
        def _group(g):
            _do_group((gbase + g) * GB)

        @pl.when(wid < XG)
        def _extra():
            _do_group((gbase + GPW) * GB)

        @pl.when(wid >= NW - TW)
        def _tail():
            row = TAIL0 + wid - (NW - TW)
            t = (wid - (NW - TW)) * B
            pltpu.sync_copy(src_hbm.at[pl.ds(row * B, B)],
                            sgrp.at[pl.ds(0, B)])
            pltpu.sync_copy(dt_hbm.at[pl.ds(t, B)], dtail)
            pltpu.async_copy(tab_hbm.at[sgrp.at[pl.ds(0, B)]], rowsa,
                             gsa).wait()
            pltpu.sync_copy(rowsa, acc.at[dtail], add=True)

        plsc.subcore_barrier()

        @pl.when(s == 0)
        def _():
            pltpu.sync_copy(acc, out_hbm.at[c])

    return agg


_agg_h = _make_agg(H)


def _inv(col):
    return jnp.where(col > 0, lax.rsqrt(jnp.maximum(col, 1.0)), 0.0)


def _mm1_body(x_ref, w_ref, deg_ref, o_ref):
    d = deg_ref[0] + deg_ref[1]
    inv_out = _inv(d[:, 0:1])
    o_ref[...] = (
        jnp.dot(x_ref[...], w_ref[...], preferred_element_type=jnp.float32)
        * inv_out
    )


def _mm2_body(a_ref, deg_ref, w_ref, o_ref):
    d = deg_ref[0] + deg_ref[1]
    inv_out = _inv(d[:, 0:1])
    inv_in = _inv(d[:, 1:2])
    h = jnp.maximum((a_ref[0] + a_ref[1]) * inv_in, 0.0)
    o_ref[...] = (
        jnp.dot(h, w_ref[...], preferred_element_type=jnp.float32) * inv_out
    )


def _fin_body(a_ref, deg_ref, o_ref):
    d = deg_ref[0] + deg_ref[1]
    inv_in = _inv(d[:, 1:2])
    o_ref[...] = (a_ref[0, :, :C] + a_ref[1, :, :C]) * inv_in


R1 = 1000
R2 = 1024


def kernel(x, edge_index, W1, W2):
    src = edge_index[0]
    dst = edge_index[1]
    src2d = src.reshape(CH, B)
    dst2d = dst.reshape(CH, B)
    stl = lax.slice(src, (TAIL0 * B,), (E,))
    dtl = lax.slice(dst, (TAIL0 * B,), (E,))

    degp = _deg_kernel(src2d, dst2d, stl, dtl)

    h1s = pl.pallas_call(
        _mm1_body,
        grid=(N // R1,),
        in_specs=[
            pl.BlockSpec((R1, F), lambda i: (i, 0)),
            pl.BlockSpec((F, H), lambda i: (0, 0)),
            pl.BlockSpec((NC, R1, 128), lambda i: (0, i, 0)),
        ],
        out_specs=pl.BlockSpec((R1, H), lambda i: (i, 0)),
        out_shape=jax.ShapeDtypeStruct((N, H), jnp.float32),
    )(x, W1, degp)

    agg1 = _agg_h(h1s, src, dst2d, dtl)

    W2p = jnp.concatenate([W2, jnp.zeros((H, H - C), jnp.float32)], axis=1)
    h2s = pl.pallas_call(
        _mm2_body,
        grid=(NPAD // R2,),
        in_specs=[
            pl.BlockSpec((NC, R2, H), lambda i: (0, i, 0)),
            pl.BlockSpec((NC, R2, 128), lambda i: (0, i, 0)),
            pl.BlockSpec((H, H), lambda i: (0, 0)),
        ],
        out_specs=pl.BlockSpec((R2, H), lambda i: (i, 0)),
        out_shape=jax.ShapeDtypeStruct((NPAD, H), jnp.float32),
    )(agg1, degp, W2p)

    agg2 = _agg_h(h2s, src, dst2d, dtl)

    outp = pl.pallas_call(
        _fin_body,
        grid=(NPAD // R2,),
        in_specs=[
            pl.BlockSpec((NC, R2, H), lambda i: (0, i, 0)),
            pl.BlockSpec((NC, R2, 128), lambda i: (0, i, 0)),
        ],
        out_specs=pl.BlockSpec((R2, C), lambda i: (i, 0)),
        out_shape=jax.ShapeDtypeStruct((NPAD, C), jnp.float32),
    )(agg2, degp)

    return outp[:N]
